# baseline (device time: 32955 ns/iter reference)
import jax
import jax.numpy as jnp
from jax import lax
from jax.experimental import pallas as pl
from jax.experimental.pallas import tpu as pltpu

N_DEV = 16
ROUNDS = 2
P = N_DEV - 1


def kernel(t, W):
    m_per, k = t.shape
    _, n = W.shape
    rows = m_per // N_DEV
    sub = rows // ROUNDS

    def body(t_ref, w_ref, out_ref, tb_ref, stage_ref,
             s1_send, s1_recv, s2_send, s2_recv):
        my = lax.axis_index("i")

        barrier_sem = pltpu.get_barrier_semaphore()
        for off in range(1, N_DEV):
            nbr = lax.rem(my + off, N_DEV)
            pl.semaphore_signal(
                barrier_sem, inc=1,
                device_id=(nbr,), device_id_type=pl.DeviceIdType.MESH,
            )
        pl.semaphore_wait(barrier_sem, P)

        tb_ref[:, :] = t_ref[:, :].astype(jnp.bfloat16)

        wb = w_ref[:, :].astype(jnp.bfloat16)

        p1 = {}
        for r in range(ROUNDS):
            for off in range(1, N_DEV):
                idx = r * P + off - 1
                dst = lax.rem(my + off, N_DEV)
                rdma = pltpu.make_async_remote_copy(
                    src_ref=tb_ref.at[pl.ds(dst * rows + r * sub, sub)],
                    dst_ref=stage_ref.at[idx],
                    send_sem=s1_send.at[idx],
                    recv_sem=s1_recv.at[idx],
                    device_id=(dst,),
                    device_id_type=pl.DeviceIdType.MESH,
                )
                rdma.start()
                p1[idx] = rdma

        p2 = {}
        for r in range(ROUNDS):
            acc = tb_ref[pl.ds(my * rows + r * sub, sub), :].astype(
                jnp.float32
            )
            for off in range(1, N_DEV):
                idx = r * P + off - 1
                p1[idx].wait_recv()
                acc = acc + stage_ref[idx, :, :].astype(jnp.float32)
            for off in range(1, N_DEV):
                p1[r * P + off - 1].wait_send()

            oc = lax.dot(
                acc.astype(jnp.bfloat16), wb,
                preferred_element_type=jnp.float32,
            )
            out_ref[pl.ds(my * rows + r * sub, sub), :] = oc.astype(
                jnp.bfloat16
            )

            for off in range(1, N_DEV):
                idx = r * P + off - 1
                dst = lax.rem(my + off, N_DEV)
                rdma = pltpu.make_async_remote_copy(
                    src_ref=out_ref.at[pl.ds(my * rows + r * sub, sub)],
                    dst_ref=out_ref.at[pl.ds(my * rows + r * sub, sub)],
                    send_sem=s2_send.at[idx],
                    recv_sem=s2_recv.at[idx],
                    device_id=(dst,),
                    device_id_type=pl.DeviceIdType.MESH,
                )
                rdma.start()
                p2[idx] = rdma

        for idx in range(ROUNDS * P):
            p2[idx].wait_recv()
        for idx in range(ROUNDS * P):
            p2[idx].wait_send()

    return pl.pallas_call(
        body,
        out_shape=jax.ShapeDtypeStruct((m_per, n), jnp.bfloat16),
        in_specs=[
            pl.BlockSpec(memory_space=pltpu.VMEM),
            pl.BlockSpec(memory_space=pltpu.VMEM),
        ],
        out_specs=pl.BlockSpec(memory_space=pltpu.VMEM),
        scratch_shapes=[
            pltpu.VMEM((m_per, k), jnp.bfloat16),
            pltpu.VMEM((ROUNDS * P, sub, k), jnp.bfloat16),
            pltpu.SemaphoreType.DMA((ROUNDS * P,)),
            pltpu.SemaphoreType.DMA((ROUNDS * P,)),
            pltpu.SemaphoreType.DMA((ROUNDS * P,)),
            pltpu.SemaphoreType.DMA((ROUNDS * P,)),
        ],
        compiler_params=pltpu.CompilerParams(collective_id=0),
    )(t, W)


# device time: 31294 ns/iter; 1.0531x vs baseline; 1.0531x over previous
import jax
import jax.numpy as jnp
from jax import lax
from jax.experimental import pallas as pl
from jax.experimental.pallas import tpu as pltpu

N_DEV = 16


def kernel(t, W):
    m_per, k = t.shape
    _, n = W.shape
    rows = m_per // N_DEV

    def body(t_ref, w_ref, out_ref, tb_ref, stage_ref,
             s1_send, s1_recv, s2_send, s2_recv):
        my = lax.axis_index("i")

        tb_ref[:, :] = t_ref[:, :].astype(jnp.bfloat16)

        barrier_sem = pltpu.get_barrier_semaphore()
        for off in range(1, N_DEV):
            nbr = lax.rem(my + off, N_DEV)
            pl.semaphore_signal(
                barrier_sem, inc=1,
                device_id=(nbr,), device_id_type=pl.DeviceIdType.MESH,
            )
        pl.semaphore_wait(barrier_sem, N_DEV - 1)

        p1 = []
        for off in range(1, N_DEV):
            dst = lax.rem(my + off, N_DEV)
            rdma = pltpu.make_async_remote_copy(
                src_ref=tb_ref.at[pl.ds(dst * rows, rows)],
                dst_ref=stage_ref.at[off - 1],
                send_sem=s1_send.at[off - 1],
                recv_sem=s1_recv.at[off - 1],
                device_id=(dst,),
                device_id_type=pl.DeviceIdType.MESH,
            )
            rdma.start()
            p1.append(rdma)

        acc = tb_ref[pl.ds(my * rows, rows), :]
        for off in range(1, N_DEV):
            p1[off - 1].wait_recv()
            acc = acc + stage_ref[off - 1, :, :]

        oc = lax.dot(
            acc,
            w_ref[:, :].astype(jnp.bfloat16),
            preferred_element_type=jnp.float32,
        )
        out_ref[pl.ds(my * rows, rows), :] = oc.astype(jnp.bfloat16)

        p2 = []
        for off in range(1, N_DEV):
            dst = lax.rem(my + off, N_DEV)
            rdma = pltpu.make_async_remote_copy(
                src_ref=out_ref.at[pl.ds(my * rows, rows)],
                dst_ref=out_ref.at[pl.ds(my * rows, rows)],
                send_sem=s2_send.at[off - 1],
                recv_sem=s2_recv.at[off - 1],
                device_id=(dst,),
                device_id_type=pl.DeviceIdType.MESH,
            )
            rdma.start()
            p2.append(rdma)

        for off in range(1, N_DEV):
            p2[off - 1].wait_recv()

        for off in range(1, N_DEV):
            p1[off - 1].wait_send()
            p2[off - 1].wait_send()

    return pl.pallas_call(
        body,
        out_shape=jax.ShapeDtypeStruct((m_per, n), jnp.bfloat16),
        in_specs=[
            pl.BlockSpec(memory_space=pltpu.VMEM),
            pl.BlockSpec(memory_space=pltpu.VMEM),
        ],
        out_specs=pl.BlockSpec(memory_space=pltpu.VMEM),
        scratch_shapes=[
            pltpu.VMEM((m_per, k), jnp.bfloat16),
            pltpu.VMEM((N_DEV - 1, rows, k), jnp.bfloat16),
            pltpu.SemaphoreType.DMA((N_DEV - 1,)),
            pltpu.SemaphoreType.DMA((N_DEV - 1,)),
            pltpu.SemaphoreType.DMA((N_DEV - 1,)),
            pltpu.SemaphoreType.DMA((N_DEV - 1,)),
        ],
        compiler_params=pltpu.CompilerParams(collective_id=0),
    )(t, W)


# device time: 31181 ns/iter; 1.0569x vs baseline; 1.0036x over previous
import jax
import jax.numpy as jnp
from jax import lax
from jax.experimental import pallas as pl
from jax.experimental.pallas import tpu as pltpu

N_DEV = 16


def kernel(t, W):
    m_per, k = t.shape
    _, n = W.shape
    rows = m_per // N_DEV

    def body(t_ref, w_ref, out_ref, tb_ref, stage_ref,
             s1_send, s1_recv, s2_send, s2_recv):
        my = lax.axis_index("i")

        barrier_sem = pltpu.get_barrier_semaphore()
        for off in range(1, N_DEV):
            nbr = lax.rem(my + off, N_DEV)
            pl.semaphore_signal(
                barrier_sem, inc=1,
                device_id=(nbr,), device_id_type=pl.DeviceIdType.MESH,
            )
        pl.semaphore_wait(barrier_sem, N_DEV - 1)

        p1 = []
        for off in range(1, N_DEV):
            dst = lax.rem(my + off, N_DEV)
            tb_ref[pl.ds(dst * rows, rows), :] = t_ref[
                pl.ds(dst * rows, rows), :
            ].astype(jnp.bfloat16)
            rdma = pltpu.make_async_remote_copy(
                src_ref=tb_ref.at[pl.ds(dst * rows, rows)],
                dst_ref=stage_ref.at[off - 1],
                send_sem=s1_send.at[off - 1],
                recv_sem=s1_recv.at[off - 1],
                device_id=(dst,),
                device_id_type=pl.DeviceIdType.MESH,
            )
            rdma.start()
            p1.append(rdma)

        acc = t_ref[pl.ds(my * rows, rows), :].astype(jnp.bfloat16)
        for off in range(1, N_DEV):
            p1[off - 1].wait_recv()
            acc = acc + stage_ref[off - 1, :, :]

        oc = lax.dot(
            acc,
            w_ref[:, :].astype(jnp.bfloat16),
            preferred_element_type=jnp.float32,
        )
        out_ref[pl.ds(my * rows, rows), :] = oc.astype(jnp.bfloat16)

        p2 = []
        for off in range(1, N_DEV):
            dst = lax.rem(my + off, N_DEV)
            rdma = pltpu.make_async_remote_copy(
                src_ref=out_ref.at[pl.ds(my * rows, rows)],
                dst_ref=out_ref.at[pl.ds(my * rows, rows)],
                send_sem=s2_send.at[off - 1],
                recv_sem=s2_recv.at[off - 1],
                device_id=(dst,),
                device_id_type=pl.DeviceIdType.MESH,
            )
            rdma.start()
            p2.append(rdma)

        for off in range(1, N_DEV):
            p2[off - 1].wait_recv()

        for off in range(1, N_DEV):
            p1[off - 1].wait_send()
            p2[off - 1].wait_send()

    return pl.pallas_call(
        body,
        out_shape=jax.ShapeDtypeStruct((m_per, n), jnp.bfloat16),
        in_specs=[
            pl.BlockSpec(memory_space=pltpu.VMEM),
            pl.BlockSpec(memory_space=pltpu.VMEM),
        ],
        out_specs=pl.BlockSpec(memory_space=pltpu.VMEM),
        scratch_shapes=[
            pltpu.VMEM((m_per, k), jnp.bfloat16),
            pltpu.VMEM((N_DEV - 1, rows, k), jnp.bfloat16),
            pltpu.SemaphoreType.DMA((N_DEV - 1,)),
            pltpu.SemaphoreType.DMA((N_DEV - 1,)),
            pltpu.SemaphoreType.DMA((N_DEV - 1,)),
            pltpu.SemaphoreType.DMA((N_DEV - 1,)),
        ],
        compiler_params=pltpu.CompilerParams(collective_id=0),
    )(t, W)
